# Initial kernel scaffold; baseline (speedup 1.0000x reference)
import functools

import jax
import jax.numpy as jnp
from jax import lax
from jax.experimental import pallas as pl
from jax.experimental.pallas import tpu as pltpu
from jax.experimental.pallas import tpu_sc as plsc

_NC = 2
_NS = 16
_NW = _NC * _NS
_CH = 128


@functools.partial(jax.jit, static_argnums=(2, 3, 4))
def _sc_embedding_lookup(idx, table, n_rows, per_w, n_ch):
    d = table.shape[1]
    mesh = plsc.VectorSubcoreMesh(core_axis_name="c", subcore_axis_name="s")

    @functools.partial(
        pl.kernel,
        out_type=jax.ShapeDtypeStruct((n_rows, d), table.dtype),
        mesh=mesh,
        scratch_types=[
            pltpu.VMEM((n_ch, _CH), jnp.int32),
            pltpu.VMEM((2, _CH, d), table.dtype),
            pltpu.SemaphoreType.DMA,
            pltpu.SemaphoreType.DMA,
        ],
    )
    def emb(idx_hbm, table_hbm, out_hbm, idx_v, rows_v, gsem, ssem):
        wid = lax.axis_index("s") * _NC + lax.axis_index("c")
        base = wid * per_w
        pltpu.sync_copy(idx_hbm.at[wid], idx_v)

        pltpu.async_copy(table_hbm.at[idx_v.at[0]], rows_v.at[0], gsem)

        @pl.loop(0, n_ch - 1)
        def _(j):
            slot = lax.rem(j, 2)
            nslot = lax.rem(j + 1, 2)
            pltpu.async_copy(
                table_hbm.at[idx_v.at[j + 1]], rows_v.at[nslot], gsem
            )
            pltpu.make_async_copy(
                table_hbm.at[idx_v.at[j]], rows_v.at[slot], gsem
            ).wait()
            pltpu.async_copy(
                rows_v.at[slot], out_hbm.at[pl.ds(base + j * _CH, _CH)], ssem
            )

        last = n_ch - 1
        lslot = last % 2
        pltpu.make_async_copy(
            table_hbm.at[idx_v.at[last]], rows_v.at[lslot], gsem
        ).wait()
        pltpu.sync_copy(
            rows_v.at[lslot], out_hbm.at[pl.ds(base + last * _CH, _CH)]
        )

        @pl.loop(0, n_ch - 1)
        def _(j):
            slot = lax.rem(j, 2)
            pltpu.make_async_copy(
                rows_v.at[slot], out_hbm.at[pl.ds(base + j * _CH, _CH)], ssem
            ).wait()

    return emb(idx, table)


def kernel(inputs, table):
    b, f = inputs.shape
    n = b * f
    per_w = n // _NW
    n_ch = per_w // _CH
    idx = inputs.reshape(_NW, n_ch, _CH).astype(jnp.int32)
    out = _sc_embedding_lookup(idx, table, n, per_w, n_ch)
    return out.reshape(b, f, table.shape[1])


# speedup vs baseline: 1.5384x; 1.5384x over previous
import functools

import jax
import jax.numpy as jnp
from jax import lax
from jax.experimental import pallas as pl
from jax.experimental.pallas import tpu as pltpu
from jax.experimental.pallas import tpu_sc as plsc

_NC = 2
_NS = 16
_NW = _NC * _NS
_CH = 128
_K = 4


@functools.partial(jax.jit, static_argnums=(2, 3, 4))
def _sc_embedding_lookup(idx, table, n_rows, per_w, n_ch):
    d = table.shape[1]
    n_mc = n_ch // _K
    mc = _K * _CH
    mesh = plsc.VectorSubcoreMesh(core_axis_name="c", subcore_axis_name="s")

    @functools.partial(
        pl.kernel,
        out_type=jax.ShapeDtypeStruct((n_rows, d), table.dtype),
        mesh=mesh,
        compiler_params=pltpu.CompilerParams(use_tc_tiling_on_sc=False),
        scratch_types=[
            pltpu.VMEM((n_ch, _CH), jnp.int32),
            pltpu.VMEM((mc, d), table.dtype),
            pltpu.SemaphoreType.DMA,
        ],
    )
    def emb(idx_hbm, table_hbm, out_hbm, idx_v, rows_v, gsem):
        wid = lax.axis_index("s") * _NC + lax.axis_index("c")
        base = wid * per_w
        pltpu.sync_copy(idx_hbm.at[wid], idx_v)

        @pl.loop(0, n_mc)
        def _(m):
            for b in range(_K):
                pltpu.async_copy(
                    table_hbm.at[idx_v.at[m * _K + b]],
                    rows_v.at[pl.ds(b * _CH, _CH)],
                    gsem,
                )
            for b in range(_K):
                pltpu.make_async_copy(
                    table_hbm.at[idx_v.at[m * _K + b]],
                    rows_v.at[pl.ds(b * _CH, _CH)],
                    gsem,
                ).wait()
            pltpu.sync_copy(rows_v, out_hbm.at[pl.ds(base + m * mc, mc)])

    return emb(idx, table)


def kernel(inputs, table):
    b, f = inputs.shape
    n = b * f
    per_w = n // _NW
    n_ch = per_w // _CH
    idx = inputs.reshape(_NW, n_ch, _CH).astype(jnp.int32)
    out = _sc_embedding_lookup(idx, table, n, per_w, n_ch)
    return out.reshape(b, f, table.shape[1])


# speedup vs baseline: 1.5728x; 1.0223x over previous
import functools

import jax
import jax.numpy as jnp
from jax import lax
from jax.experimental import pallas as pl
from jax.experimental.pallas import tpu as pltpu
from jax.experimental.pallas import tpu_sc as plsc

_NC = 2
_NS = 16
_NW = _NC * _NS
_CH = 128
_K = 4


@functools.partial(jax.jit, static_argnums=(2, 3, 4))
def _sc_embedding_lookup(idx, table, n_rows, per_w, n_ch):
    d = table.shape[1]
    n_mc = n_ch // _K
    mc = _K * _CH
    mesh = plsc.VectorSubcoreMesh(core_axis_name="c", subcore_axis_name="s")

    n_pair = n_mc // 2

    @functools.partial(
        pl.kernel,
        out_type=jax.ShapeDtypeStruct((n_rows, d), table.dtype),
        mesh=mesh,
        compiler_params=pltpu.CompilerParams(use_tc_tiling_on_sc=False),
        scratch_types=[
            pltpu.VMEM((n_ch, _CH), jnp.int32),
            pltpu.VMEM((2, mc, d), table.dtype),
            pltpu.SemaphoreType.DMA,
            pltpu.SemaphoreType.DMA,
            pltpu.SemaphoreType.DMA,
            pltpu.SemaphoreType.DMA,
        ],
    )
    def emb(idx_hbm, table_hbm, out_hbm, idx_v, rows_v, g0, g1, s0, s1):
        wid = lax.axis_index("s") * _NC + lax.axis_index("c")
        base = wid * per_w
        gsems = (g0, g1)
        ssems = (s0, s1)
        pltpu.sync_copy(idx_hbm.at[wid], idx_v)

        def fire(m, slot):
            for b in range(_K):
                pltpu.async_copy(
                    table_hbm.at[idx_v.at[m * _K + b]],
                    rows_v.at[slot].at[pl.ds(b * _CH, _CH)],
                    gsems[slot],
                )

        def drain(m, slot):
            for b in range(_K):
                pltpu.make_async_copy(
                    table_hbm.at[idx_v.at[m * _K + b]],
                    rows_v.at[slot].at[pl.ds(b * _CH, _CH)],
                    gsems[slot],
                ).wait()

        def store(m, slot):
            pltpu.async_copy(
                rows_v.at[slot], out_hbm.at[pl.ds(base + m * mc, mc)],
                ssems[slot],
            )

        def wait_store(m, slot):
            pltpu.make_async_copy(
                rows_v.at[slot], out_hbm.at[pl.ds(base + m * mc, mc)],
                ssems[slot],
            ).wait()

        fire(0, 0)

        @pl.loop(0, n_pair)
        def _(p):
            m0 = 2 * p
            @pl.when(p >= 1)
            def _():
                wait_store(m0 - 1, 1)

            fire(m0 + 1, 1)
            drain(m0, 0)
            store(m0, 0)
            wait_store(m0, 0)

            @pl.when(p < n_pair - 1)
            def _():
                fire(m0 + 2, 0)

            drain(m0 + 1, 1)
            store(m0 + 1, 1)

        wait_store(n_mc - 1, 1)

    return emb(idx, table)


def kernel(inputs, table):
    b, f = inputs.shape
    n = b * f
    per_w = n // _NW
    n_ch = per_w // _CH
    idx = inputs.reshape(_NW, n_ch, _CH).astype(jnp.int32)
    out = _sc_embedding_lookup(idx, table, n, per_w, n_ch)
    return out.reshape(b, f, table.shape[1])


# speedup vs baseline: 1.5763x; 1.0022x over previous
import functools

import jax
import jax.numpy as jnp
from jax import lax
from jax.experimental import pallas as pl
from jax.experimental.pallas import tpu as pltpu
from jax.experimental.pallas import tpu_sc as plsc

_NC = 2
_NS = 16
_NW = _NC * _NS
_CH = 128
_K = 8


@functools.partial(jax.jit, static_argnums=(2, 3, 4))
def _sc_embedding_lookup(idx, table, n_rows, per_w, n_ch):
    d = table.shape[1]
    n_mc = n_ch // _K
    mc = _K * _CH
    mesh = plsc.VectorSubcoreMesh(core_axis_name="c", subcore_axis_name="s")

    n_pair = n_mc // 2

    @functools.partial(
        pl.kernel,
        out_type=jax.ShapeDtypeStruct((n_rows, d), table.dtype),
        mesh=mesh,
        compiler_params=pltpu.CompilerParams(use_tc_tiling_on_sc=False),
        scratch_types=[
            pltpu.VMEM((n_ch, _CH), jnp.int32),
            pltpu.VMEM((2, mc, d), table.dtype),
            pltpu.SemaphoreType.DMA,
            pltpu.SemaphoreType.DMA,
            pltpu.SemaphoreType.DMA,
            pltpu.SemaphoreType.DMA,
        ],
    )
    def emb(idx_hbm, table_hbm, out_hbm, idx_v, rows_v, g0, g1, s0, s1):
        wid = lax.axis_index("s") * _NC + lax.axis_index("c")
        base = wid * per_w
        gsems = (g0, g1)
        ssems = (s0, s1)
        pltpu.sync_copy(idx_hbm.at[wid], idx_v)

        def fire(m, slot):
            for b in range(_K):
                pltpu.async_copy(
                    table_hbm.at[idx_v.at[m * _K + b]],
                    rows_v.at[slot].at[pl.ds(b * _CH, _CH)],
                    gsems[slot],
                )

        def drain(m, slot):
            for b in range(_K):
                pltpu.make_async_copy(
                    table_hbm.at[idx_v.at[m * _K + b]],
                    rows_v.at[slot].at[pl.ds(b * _CH, _CH)],
                    gsems[slot],
                ).wait()

        def store(m, slot):
            pltpu.async_copy(
                rows_v.at[slot], out_hbm.at[pl.ds(base + m * mc, mc)],
                ssems[slot],
            )

        def wait_store(m, slot):
            pltpu.make_async_copy(
                rows_v.at[slot], out_hbm.at[pl.ds(base + m * mc, mc)],
                ssems[slot],
            ).wait()

        fire(0, 0)

        @pl.loop(0, n_pair)
        def _(p):
            m0 = 2 * p
            @pl.when(p >= 1)
            def _():
                wait_store(m0 - 1, 1)

            fire(m0 + 1, 1)
            drain(m0, 0)
            store(m0, 0)
            wait_store(m0, 0)

            @pl.when(m0 + 2 < n_mc)
            def _():
                fire(m0 + 2, 0)

            drain(m0 + 1, 1)
            store(m0 + 1, 1)

        if n_mc % 2:
            mt = n_mc - 1
            wait_store(mt - 1, 1)
            drain(mt, 0)
            store(mt, 0)
            wait_store(mt, 0)
        else:
            wait_store(n_mc - 1, 1)

    return emb(idx, table)


def kernel(inputs, table):
    b, f = inputs.shape
    n = b * f
    per_w = n // _NW
    n_ch = per_w // _CH
    idx = inputs.reshape(_NW, n_ch, _CH).astype(jnp.int32)
    out = _sc_embedding_lookup(idx, table, n, per_w, n_ch)
    return out.reshape(b, f, table.shape[1])
